# Initial kernel scaffold; baseline (speedup 1.0000x reference)
import functools

import jax
import jax.numpy as jnp
from jax import lax
from jax.experimental import pallas as pl
from jax.experimental.pallas import tpu as pltpu
from jax.experimental.pallas import tpu_sc as plsc

_L = 16


def _make_mask_kernel(batch, seq):
    mesh = plsc.VectorSubcoreMesh(core_axis_name="c", subcore_axis_name="s")

    @functools.partial(
        pl.kernel,
        mesh=mesh,
        out_type=jax.ShapeDtypeStruct((batch, seq), jnp.float32),
        scratch_types=[
            pltpu.VMEM((seq,), jnp.int32),
            pltpu.VMEM((_L,), jnp.int32),
            pltpu.VMEM((seq,), jnp.float32),
        ],
    )
    def mask_kernel(orders_hbm, nb_hbm, out_hbm, idx_v, n_v, row_v):
        wid = lax.axis_index("s") * 2 + lax.axis_index("c")

        @pl.when(wid < batch)
        def _():
            pltpu.sync_copy(orders_hbm.at[wid], idx_v)
            pltpu.sync_copy(nb_hbm.at[wid], n_v)
            nvec = n_v[...]

            def body(t, carry):
                j0 = t * _L
                idx16 = idx_v[pl.ds(j0, _L)]
                jvec = lax.iota(jnp.int32, 16) + j0
                vals = jnp.where(jvec < nvec, 1.0, 0.0)
                plsc.store_scatter(row_v, [idx16], vals)
                return carry

            lax.fori_loop(0, seq // _L, body, 0)
            pltpu.sync_copy(row_v, out_hbm.at[wid])

    return mask_kernel


def kernel(patches, orders):
    batch, seq, _ = patches.shape
    n = jax.random.randint(jax.random.key(42), (batch,), 1, seq + 1)
    nb = jnp.broadcast_to(n.astype(jnp.int32)[:, None], (batch, _L))
    idx = orders.astype(jnp.int32)
    return _make_mask_kernel(batch, seq)(idx, nb)


# speedup vs baseline: 2.8317x; 2.8317x over previous
import functools

import jax
import jax.numpy as jnp
from jax import lax
from jax.experimental import pallas as pl
from jax.experimental.pallas import tpu as pltpu
from jax.experimental.pallas import tpu_sc as plsc

_L = 16


def _make_mask_kernel(batch, seq):
    mesh = plsc.VectorSubcoreMesh(core_axis_name="c", subcore_axis_name="s")

    @functools.partial(
        pl.kernel,
        mesh=mesh,
        out_type=jax.ShapeDtypeStruct((batch, seq), jnp.float32),
        compiler_params=pltpu.CompilerParams(needs_layout_passes=False),
        scratch_types=[
            pltpu.VMEM((seq,), jnp.int32),
            pltpu.VMEM((_L,), jnp.int32),
            pltpu.VMEM((seq,), jnp.float32),
        ],
    )
    def mask_kernel(orders_hbm, nb_hbm, out_hbm, idx_v, n_v, row_v):
        wid = lax.axis_index("s") * 2 + lax.axis_index("c")

        @pl.when(wid < batch)
        def _():
            pltpu.sync_copy(orders_hbm.at[wid], idx_v)
            pltpu.sync_copy(nb_hbm.at[wid], n_v)
            nvec = n_v[...]

            def body(t, carry):
                j0 = t * _L
                idx16 = idx_v[pl.ds(j0, _L)]
                jvec = lax.iota(jnp.int32, 16) + j0
                vals = jnp.where(jvec < nvec, 1.0, 0.0)
                plsc.store_scatter(row_v, [idx16], vals)
                return carry

            lax.fori_loop(0, seq // _L, body, 0)
            pltpu.sync_copy(row_v, out_hbm.at[wid])

    return mask_kernel


def kernel(patches, orders):
    batch, seq, _ = patches.shape
    n = jax.random.randint(jax.random.key(42), (batch,), 1, seq + 1)
    nb = jnp.broadcast_to(n.astype(jnp.int32)[:, None], (batch, _L))
    idx = orders.astype(jnp.int32)
    return _make_mask_kernel(batch, seq)(idx, nb)


# speedup vs baseline: 2.9630x; 1.0464x over previous
import functools

import jax
import jax.numpy as jnp
from jax import lax
from jax.experimental import pallas as pl
from jax.experimental.pallas import tpu as pltpu
from jax.experimental.pallas import tpu_sc as plsc

_L = 16


def _make_mask_kernel(batch, seq):
    mesh = plsc.VectorSubcoreMesh(core_axis_name="c", subcore_axis_name="s")

    @functools.partial(
        pl.kernel,
        mesh=mesh,
        out_type=jax.ShapeDtypeStruct((batch, seq), jnp.float32),
        compiler_params=pltpu.CompilerParams(needs_layout_passes=False),
        scratch_types=[
            pltpu.VMEM((seq,), jnp.int32),
            pltpu.VMEM((_L,), jnp.int32),
            pltpu.VMEM((seq,), jnp.float32),
        ],
    )
    def mask_kernel(orders_hbm, nb_hbm, out_hbm, idx_v, n_v, row_v):
        wid = lax.axis_index("s") * 2 + lax.axis_index("c")

        @pl.when(wid < batch)
        def _():
            pltpu.sync_copy(orders_hbm.at[wid], idx_v)
            pltpu.sync_copy(nb_hbm.at[wid], n_v)
            nvec = n_v[...]
            jbase = lax.iota(jnp.int32, 16)

            @plsc.parallel_loop(0, seq, step=_L, unroll=8)
            def _body(j0):
                idx16 = idx_v[pl.ds(j0, _L)]
                vals = jnp.where(jbase + j0 < nvec, 1.0, 0.0)
                plsc.store_scatter(row_v, [idx16], vals)

            pltpu.sync_copy(row_v, out_hbm.at[wid])

    return mask_kernel


def kernel(patches, orders):
    batch, seq, _ = patches.shape
    n = jax.random.randint(jax.random.key(42), (batch,), 1, seq + 1)
    nb = jnp.broadcast_to(n.astype(jnp.int32)[:, None], (batch, _L))
    idx = orders.astype(jnp.int32)
    return _make_mask_kernel(batch, seq)(idx, nb)
